# baseline (speedup 1.0000x reference)
import functools

import jax
import jax.numpy as jnp
import numpy as np
from jax import lax
from jax.experimental import pallas as pl
from jax.experimental.pallas import tpu as pltpu
from jax.experimental.pallas import tpu_sc as plsc

_NPOINT = 512
_NSAMPLE = 32
_EPS = 1e-5
_FPAD = 128


def _fps_kernel(xs_ref, ys_ref, zs_ref, idx_ref, dist_ref):
    B, N = xs_ref.shape
    lane = jax.lax.broadcasted_iota(jnp.int32, (B, N), 1)
    ii = jax.lax.broadcasted_iota(jnp.int32, (B, B), 0)
    jj = jax.lax.broadcasted_iota(jnp.int32, (B, B), 1)
    xs = xs_ref[...]
    ys = ys_ref[...]
    zs = zs_ref[...]
    dist_ref[...] = jnp.full((B, N), 1e10, jnp.float32)

    def step(k, far):
        farf = far.astype(jnp.float32)
        row = jnp.sum(jnp.where(ii == jj, farf, 0.0), axis=0, keepdims=True)
        idx_ref[pl.ds(k, 1), :] = row.astype(jnp.int32)
        m = lane == far
        cx = jnp.sum(jnp.where(m, xs, 0.0), axis=1, keepdims=True)
        cy = jnp.sum(jnp.where(m, ys, 0.0), axis=1, keepdims=True)
        cz = jnp.sum(jnp.where(m, zs, 0.0), axis=1, keepdims=True)
        d = (xs - cx) ** 2 + (ys - cy) ** 2 + (zs - cz) ** 2
        nd = jnp.minimum(dist_ref[...], d)
        dist_ref[...] = nd
        mx = jnp.max(nd, axis=1, keepdims=True)
        sel = jnp.where(nd == mx, lane, N)
        return jnp.min(sel, axis=1, keepdims=True).astype(jnp.int32)

    jax.lax.fori_loop(0, _NPOINT, step, jnp.zeros((B, 1), jnp.int32))


def _fps(xyz):
    B, N, _ = xyz.shape
    xs = xyz[..., 0]
    ys = xyz[..., 1]
    zs = xyz[..., 2]
    idx_t = pl.pallas_call(
        _fps_kernel,
        out_shape=jax.ShapeDtypeStruct((_NPOINT, B), jnp.int32),
        scratch_shapes=[pltpu.VMEM((B, N), jnp.float32)],
    )(xs, ys, zs)
    return idx_t.T


def _sc_topk(d2f):
    ROWS, N = d2f.shape
    NW = 32
    rows_per_w = ROWS // NW
    NCH = N // 16
    CAP = N + 16
    mesh = plsc.VectorSubcoreMesh(core_axis_name="c", subcore_axis_name="s")
    INF = jnp.float32(jnp.inf)
    IMAX = jnp.int32(2**31 - 1)

    @functools.partial(
        pl.kernel,
        mesh=mesh,
        compiler_params=pltpu.CompilerParams(needs_layout_passes=False),
        out_type=jax.ShapeDtypeStruct((ROWS * _NSAMPLE,), jnp.int32),
        scratch_types=[
            pltpu.VMEM((N,), jnp.float32),
            pltpu.VMEM((CAP,), jnp.float32),
            pltpu.VMEM((CAP,), jnp.int32),
            pltpu.VMEM((_NSAMPLE,), jnp.int32),
        ],
    )
    def k(d2_hbm, out_hbm, rowb, cv, ci, wv):
        wid = lax.axis_index("s") * 2 + lax.axis_index("c")
        lane16 = lax.broadcasted_iota(jnp.int32, (16,), 0)

        def row_body(i, carry):
            grow = wid * rows_per_w + i
            boff = (grow // _NPOINT) * N
            pltpu.sync_copy(d2_hbm.at[grow], rowb)

            def tau_body(c, mm):
                g0, g1, g2, g3 = mm
                return (jnp.minimum(g0, rowb[pl.ds(c * 64, 16)]),
                        jnp.minimum(g1, rowb[pl.ds(c * 64 + 16, 16)]),
                        jnp.minimum(g2, rowb[pl.ds(c * 64 + 32, 16)]),
                        jnp.minimum(g3, rowb[pl.ds(c * 64 + 48, 16)]))

            finf = jnp.full((16,), INF, jnp.float32)
            g0, g1, g2, g3 = lax.fori_loop(0, N // 64, tau_body,
                                           (finf, finf, finf, finf))
            tau = jnp.max(jnp.maximum(jnp.maximum(g0, g1),
                                      jnp.maximum(g2, g3)))

            def comp_body(c, cnt):
                v0 = rowb[pl.ds(c * 32, 16)]
                v1 = rowb[pl.ds(c * 32 + 16, 16)]
                m0 = v0 <= tau
                m1 = v1 <= tau
                pc0 = plsc.all_reduce_population_count(m0)[0]
                pc1 = plsc.all_reduce_population_count(m1)[0]
                plsc.store_compressed(cv.at[pl.ds(cnt, 16)], v0, mask=m0)
                plsc.store_compressed(ci.at[pl.ds(cnt, 16)],
                                      lane16 + c * 32, mask=m0)
                c1 = cnt + pc0
                plsc.store_compressed(cv.at[pl.ds(c1, 16)], v1, mask=m1)
                plsc.store_compressed(ci.at[pl.ds(c1, 16)],
                                      lane16 + c * 32 + 16, mask=m1)
                return c1 + pc1

            cnt = lax.fori_loop(0, NCH // 2, comp_body, jnp.int32(0))
            cv[pl.ds(cnt, 16)] = jnp.full((16,), INF, jnp.float32)
            nv = (cnt + 15) // 16

            def ext_body(t, carry2):
                w0, w1, pm, pwin = carry2

                def scan_body(j, mm):
                    mv, mi = mm
                    v = cv[pl.ds(j * 16, 16)]
                    ii = ci[pl.ds(j * 16, 16)]
                    hit = (v == pm) & (ii == pwin)
                    v = jnp.where(hit, INF, v)
                    cv[pl.ds(j * 16, 16)] = v
                    sel = v < mv
                    return jnp.where(sel, v, mv), jnp.where(sel, ii, mi)

                mv, mi = lax.fori_loop(0, nv, scan_body,
                                       (jnp.full((16,), INF, jnp.float32),
                                        jnp.zeros((16,), jnp.int32)))
                m = jnp.min(mv)
                win = jnp.min(jnp.where(mv == m, mi, IMAX))
                hitlane = lane16 == (t & 15)
                w0 = jnp.where(hitlane & (t < 16), win, w0)
                w1 = jnp.where(hitlane & (t >= 16), win, w1)
                return w0, w1, m, win

            w0, w1, _, _ = lax.fori_loop(
                0, _NSAMPLE, ext_body,
                (jnp.zeros((16,), jnp.int32), jnp.zeros((16,), jnp.int32),
                 -INF, jnp.int32(-1)))
            wv[pl.ds(0, 16)] = w0 + boff
            wv[pl.ds(16, 16)] = w1 + boff
            pltpu.sync_copy(wv, out_hbm.at[pl.ds(grow * _NSAMPLE, _NSAMPLE)])
            return carry

        lax.fori_loop(0, rows_per_w, row_body, 0)

    return k(d2f)


def _sc_gather(table, fidx):
    R = fidx.shape[0]
    NW = 32
    per_w = R // NW
    CH = 256
    n_chunks = per_w // CH
    mesh = plsc.VectorSubcoreMesh(core_axis_name="c", subcore_axis_name="s")

    @functools.partial(
        pl.kernel,
        mesh=mesh,
        out_type=jax.ShapeDtypeStruct((R, _FPAD), jnp.float32),
        scratch_types=[
            pltpu.VMEM((CH,), jnp.int32),
            pltpu.VMEM((CH, _FPAD), jnp.float32),
            pltpu.SemaphoreType.DMA,
        ],
    )
    def k(tab_hbm, idx_hbm, out_hbm, idx_v, rows_v, sem):
        wid = lax.axis_index("s") * 2 + lax.axis_index("c")
        base = wid * per_w

        def body(i, carry):
            off = base + i * CH
            pltpu.sync_copy(idx_hbm.at[pl.ds(off, CH)], idx_v)
            pltpu.async_copy(tab_hbm.at[idx_v], rows_v, sem).wait()
            pltpu.sync_copy(rows_v, out_hbm.at[pl.ds(off, CH)])
            return carry

        lax.fori_loop(0, n_chunks, body, 0)

    return k(table, fidx)


def _l1_kernel(x_ref, sam_ref, mc_ref, w_ref, b_ref, y_ref, s_ref, ss_ref):
    b = pl.program_id(0)
    sam = jnp.concatenate([sam_ref[0]] * _NSAMPLE, axis=1)
    y = jnp.dot(w_ref[...], x_ref[0] - sam,
                preferred_element_type=jnp.float32)
    y = y + b_ref[...]
    y_ref[0] = y

    @pl.when(b == 0)
    def _():
        s_ref[...] = jnp.zeros_like(s_ref)
        ss_ref[...] = jnp.zeros_like(ss_ref)

    mc = mc_ref[...]
    s_ref[...] += jnp.dot(y, mc, preferred_element_type=jnp.float32)
    ss_ref[...] += jnp.dot(y * y, mc, preferred_element_type=jnp.float32)


def _l2_kernel(y1_ref, s_ref, ss_ref, mc_ref, g_ref, be_ref, w_ref, b_ref,
               y2_ref, s2_ref, ss2_ref, *, m):
    b = pl.program_id(0)
    mean = s_ref[...] / m
    var = ss_ref[...] / m - mean * mean
    scale = g_ref[...] / jnp.sqrt(var + _EPS)
    shift = be_ref[...] - mean * scale
    h = jnp.maximum(y1_ref[0] * scale + shift, 0.0)
    y = jnp.dot(w_ref[...], h, preferred_element_type=jnp.float32)
    y = y + b_ref[...]
    y2_ref[0] = y

    @pl.when(b == 0)
    def _():
        s2_ref[...] = jnp.zeros_like(s2_ref)
        ss2_ref[...] = jnp.zeros_like(ss2_ref)

    mc = mc_ref[...]
    s2_ref[...] += jnp.dot(y, mc, preferred_element_type=jnp.float32)
    ss2_ref[...] += jnp.dot(y * y, mc, preferred_element_type=jnp.float32)


def _l3_kernel(y2_ref, s_ref, ss_ref, mr_ref, g_ref, be_ref, out_ref, *, m, nw):
    mean = s_ref[...] / m
    var = ss_ref[...] / m - mean * mean
    scale = g_ref[...] / jnp.sqrt(var + _EPS)
    shift = be_ref[...] - mean * scale
    h = jnp.maximum(y2_ref[0] * scale + shift, 0.0)
    h = jnp.where(mr_ref[...] > 0.0, h, -jnp.inf)
    for s in range(_NSAMPLE):
        out_ref[0, :, pl.ds(s, 1)] = jnp.max(
            h[:, s * nw:(s + 1) * nw], axis=1, keepdims=True)


def _mlp(x, sam_pad, maskw, W1, b1, g1, be1, W2, b2, g2, be2):
    B, IC, HW = x.shape
    O1 = W1.shape[0]
    O2 = W2.shape[0]
    m = float(B * _NSAMPLE * 67)

    col = lambda v: v[:, None]
    mask_c = maskw[:, None]
    mask_r = maskw[None, :]

    y1, s1, ss1 = pl.pallas_call(
        _l1_kernel,
        grid=(B,),
        in_specs=[
            pl.BlockSpec((1, IC, HW), lambda b: (b, 0, 0)),
            pl.BlockSpec((1, IC, _FPAD), lambda b: (b, 0, 0)),
            pl.BlockSpec((HW, 1), lambda b: (0, 0)),
            pl.BlockSpec((O1, IC), lambda b: (0, 0)),
            pl.BlockSpec((O1, 1), lambda b: (0, 0)),
        ],
        out_specs=[
            pl.BlockSpec((1, O1, HW), lambda b: (b, 0, 0)),
            pl.BlockSpec((O1, 1), lambda b: (0, 0)),
            pl.BlockSpec((O1, 1), lambda b: (0, 0)),
        ],
        out_shape=[
            jax.ShapeDtypeStruct((B, O1, HW), jnp.float32),
            jax.ShapeDtypeStruct((O1, 1), jnp.float32),
            jax.ShapeDtypeStruct((O1, 1), jnp.float32),
        ],
    )(x, sam_pad, mask_c, W1, col(b1))

    y2, s2, ss2 = pl.pallas_call(
        functools.partial(_l2_kernel, m=m),
        grid=(B,),
        in_specs=[
            pl.BlockSpec((1, O1, HW), lambda b: (b, 0, 0)),
            pl.BlockSpec((O1, 1), lambda b: (0, 0)),
            pl.BlockSpec((O1, 1), lambda b: (0, 0)),
            pl.BlockSpec((HW, 1), lambda b: (0, 0)),
            pl.BlockSpec((O1, 1), lambda b: (0, 0)),
            pl.BlockSpec((O1, 1), lambda b: (0, 0)),
            pl.BlockSpec((O2, O1), lambda b: (0, 0)),
            pl.BlockSpec((O2, 1), lambda b: (0, 0)),
        ],
        out_specs=[
            pl.BlockSpec((1, O2, HW), lambda b: (b, 0, 0)),
            pl.BlockSpec((O2, 1), lambda b: (0, 0)),
            pl.BlockSpec((O2, 1), lambda b: (0, 0)),
        ],
        out_shape=[
            jax.ShapeDtypeStruct((B, O2, HW), jnp.float32),
            jax.ShapeDtypeStruct((O2, 1), jnp.float32),
            jax.ShapeDtypeStruct((O2, 1), jnp.float32),
        ],
    )(y1, s1, ss1, mask_c, col(g1), col(be1), W2, col(b2))

    out = pl.pallas_call(
        functools.partial(_l3_kernel, m=m, nw=_FPAD),
        grid=(B,),
        in_specs=[
            pl.BlockSpec((1, O2, HW), lambda b: (b, 0, 0)),
            pl.BlockSpec((O2, 1), lambda b: (0, 0)),
            pl.BlockSpec((O2, 1), lambda b: (0, 0)),
            pl.BlockSpec((1, HW), lambda b: (0, 0)),
            pl.BlockSpec((O2, 1), lambda b: (0, 0)),
            pl.BlockSpec((O2, 1), lambda b: (0, 0)),
        ],
        out_specs=pl.BlockSpec((1, O2, _NSAMPLE), lambda b: (b, 0, 0)),
        out_shape=jax.ShapeDtypeStruct((B, O2, _NSAMPLE), jnp.float32),
    )(y2, s2, ss2, mask_r, col(g2), col(be2))
    return out


def kernel(xyz, points, W1, b1, g1, be1, W2, b2, g2, be2):
    B, N, C = points.shape
    fps_idx = _fps(xyz)
    bidx = jnp.arange(B)[:, None]
    sampled = xyz[bidx, fps_idx]

    d2 = (jnp.sum(sampled ** 2, -1)[:, :, None] + jnp.sum(xyz ** 2, -1)[:, None, :]
          - 2.0 * jnp.einsum('bpd,bnd->bpn', sampled, xyz))
    fidx = _sc_topk(d2.reshape(B * _NPOINT, N))

    xyz_pad = jnp.pad(xyz, ((0, 0), (0, 0), (0, 13)))
    pts_pad = jnp.pad(points, ((0, 0), (0, 0), (0, _FPAD - 16 - C)))
    table = jnp.concatenate([xyz_pad, pts_pad], axis=2).reshape(B * N, _FPAD)
    rows = _sc_gather(table, fidx)
    x = rows.reshape(B, _NPOINT, _NSAMPLE * _FPAD)

    sam_pad = jnp.pad(sampled, ((0, 0), (0, 0), (0, _FPAD - 3)))
    w = jnp.arange(_FPAD)
    maskw = jnp.tile(((w < 3) | ((w >= 16) & (w < 16 + C))).astype(jnp.float32),
                     _NSAMPLE)

    new_points = _mlp(x, sam_pad, maskw, W1, b1, g1, be1, W2, b2, g2, be2)
    return sampled, new_points


# speedup vs baseline: 1.1194x; 1.1194x over previous
import functools

import jax
import jax.numpy as jnp
import numpy as np
from jax import lax
from jax.experimental import pallas as pl
from jax.experimental.pallas import tpu as pltpu
from jax.experimental.pallas import tpu_sc as plsc

_NPOINT = 512
_NSAMPLE = 32
_EPS = 1e-5
_FPAD = 128


def _fps_kernel(xs_ref, ys_ref, zs_ref, idx_ref, dist_ref):
    B, N = xs_ref.shape
    lane = jax.lax.broadcasted_iota(jnp.int32, (B, N), 1)
    ii = jax.lax.broadcasted_iota(jnp.int32, (B, B), 0)
    jj = jax.lax.broadcasted_iota(jnp.int32, (B, B), 1)
    xs = xs_ref[...]
    ys = ys_ref[...]
    zs = zs_ref[...]
    dist_ref[...] = jnp.full((B, N), 1e10, jnp.float32)

    def step(k, far):
        farf = far.astype(jnp.float32)
        row = jnp.sum(jnp.where(ii == jj, farf, 0.0), axis=0, keepdims=True)
        idx_ref[pl.ds(k, 1), :] = row.astype(jnp.int32)
        m = lane == far
        cx = jnp.sum(jnp.where(m, xs, 0.0), axis=1, keepdims=True)
        cy = jnp.sum(jnp.where(m, ys, 0.0), axis=1, keepdims=True)
        cz = jnp.sum(jnp.where(m, zs, 0.0), axis=1, keepdims=True)
        d = (xs - cx) ** 2 + (ys - cy) ** 2 + (zs - cz) ** 2
        nd = jnp.minimum(dist_ref[...], d)
        dist_ref[...] = nd
        mx = jnp.max(nd, axis=1, keepdims=True)
        sel = jnp.where(nd == mx, lane, N)
        return jnp.min(sel, axis=1, keepdims=True).astype(jnp.int32)

    jax.lax.fori_loop(0, _NPOINT, step, jnp.zeros((B, 1), jnp.int32))


def _fps(xyz):
    B, N, _ = xyz.shape
    xs = xyz[..., 0]
    ys = xyz[..., 1]
    zs = xyz[..., 2]
    idx_t = pl.pallas_call(
        _fps_kernel,
        out_shape=jax.ShapeDtypeStruct((_NPOINT, B), jnp.int32),
        scratch_shapes=[pltpu.VMEM((B, N), jnp.float32)],
    )(xs, ys, zs)
    return idx_t.T


def _sc_topk(d2f):
    ROWS, N = d2f.shape
    NW = 32
    rows_per_w = ROWS // NW
    NCH = N // 16
    CAP = N + 16
    mesh = plsc.VectorSubcoreMesh(core_axis_name="c", subcore_axis_name="s")
    INF = jnp.float32(jnp.inf)
    IMAX = jnp.int32(2**31 - 1)

    @functools.partial(
        pl.kernel,
        mesh=mesh,
        compiler_params=pltpu.CompilerParams(needs_layout_passes=False),
        out_type=jax.ShapeDtypeStruct((ROWS * _NSAMPLE,), jnp.int32),
        scratch_types=[
            pltpu.VMEM((N,), jnp.float32),
            pltpu.VMEM((CAP,), jnp.float32),
            pltpu.VMEM((CAP,), jnp.int32),
            pltpu.VMEM((_NSAMPLE,), jnp.int32),
        ],
    )
    def k(d2_hbm, out_hbm, rowb, cv, ci, wv):
        wid = lax.axis_index("s") * 2 + lax.axis_index("c")
        lane16 = lax.broadcasted_iota(jnp.int32, (16,), 0)

        def row_body(i, carry):
            grow = wid * rows_per_w + i
            boff = (grow // _NPOINT) * N
            pltpu.sync_copy(d2_hbm.at[grow], rowb)

            def tau_body(c, mm):
                ga, gb = mm
                va = rowb[pl.ds(c * 32, 16)]
                vb = rowb[pl.ds(c * 32 + 16, 16)]
                return jnp.minimum(ga, va), jnp.minimum(gb, vb)

            ga, gb = lax.fori_loop(0, N // 32, tau_body,
                                   (jnp.full((16,), INF, jnp.float32),
                                    jnp.full((16,), INF, jnp.float32)))
            tau = jnp.max(jnp.maximum(ga, gb))

            def comp_body(c, cnt):
                v = rowb[pl.ds(c * 16, 16)]
                msk = v <= tau
                pc = plsc.all_reduce_population_count(msk)[0]
                plsc.store_compressed(cv.at[pl.ds(cnt, 16)], v, mask=msk)
                plsc.store_compressed(ci.at[pl.ds(cnt, 16)],
                                      lane16 + c * 16, mask=msk)
                return cnt + pc

            cnt = lax.fori_loop(0, NCH, comp_body, jnp.int32(0))
            cv[pl.ds(cnt, 16)] = jnp.full((16,), INF, jnp.float32)
            nv = (cnt + 15) // 16

            def ext_body(t, carry2):
                w0, w1, pm, pwin = carry2

                def scan_body(j, mm):
                    mv, mi = mm
                    v = cv[pl.ds(j * 16, 16)]
                    ii = ci[pl.ds(j * 16, 16)]
                    hit = (v == pm) & (ii == pwin)
                    v = jnp.where(hit, INF, v)
                    cv[pl.ds(j * 16, 16)] = v
                    sel = v < mv
                    return jnp.where(sel, v, mv), jnp.where(sel, ii, mi)

                mv, mi = lax.fori_loop(0, nv, scan_body,
                                       (jnp.full((16,), INF, jnp.float32),
                                        jnp.zeros((16,), jnp.int32)))
                m = jnp.min(mv)
                win = jnp.min(jnp.where(mv == m, mi, IMAX))
                hitlane = lane16 == (t & 15)
                w0 = jnp.where(hitlane & (t < 16), win, w0)
                w1 = jnp.where(hitlane & (t >= 16), win, w1)
                return w0, w1, m, win

            w0, w1, _, _ = lax.fori_loop(
                0, _NSAMPLE, ext_body,
                (jnp.zeros((16,), jnp.int32), jnp.zeros((16,), jnp.int32),
                 -INF, jnp.int32(-1)))
            wv[pl.ds(0, 16)] = w0 + boff
            wv[pl.ds(16, 16)] = w1 + boff
            pltpu.sync_copy(wv, out_hbm.at[pl.ds(grow * _NSAMPLE, _NSAMPLE)])
            return carry

        lax.fori_loop(0, rows_per_w, row_body, 0)

    return k(d2f)


def _sc_gather(table, fidx):
    R = fidx.shape[0]
    NW = 32
    per_w = R // NW
    CH = 256
    n_chunks = per_w // CH
    mesh = plsc.VectorSubcoreMesh(core_axis_name="c", subcore_axis_name="s")

    @functools.partial(
        pl.kernel,
        mesh=mesh,
        out_type=jax.ShapeDtypeStruct((R, _FPAD), jnp.float32),
        scratch_types=[
            pltpu.VMEM((CH,), jnp.int32),
            pltpu.VMEM((CH, _FPAD), jnp.float32),
            pltpu.SemaphoreType.DMA,
        ],
    )
    def k(tab_hbm, idx_hbm, out_hbm, idx_v, rows_v, sem):
        wid = lax.axis_index("s") * 2 + lax.axis_index("c")
        base = wid * per_w

        def body(i, carry):
            off = base + i * CH
            pltpu.sync_copy(idx_hbm.at[pl.ds(off, CH)], idx_v)
            pltpu.async_copy(tab_hbm.at[idx_v], rows_v, sem).wait()
            pltpu.sync_copy(rows_v, out_hbm.at[pl.ds(off, CH)])
            return carry

        lax.fori_loop(0, n_chunks, body, 0)

    return k(table, fidx)


def _l1_kernel(x_ref, sam_ref, mc_ref, w_ref, b_ref, y_ref, s_ref, ss_ref):
    b = pl.program_id(0)
    sam = jnp.concatenate([sam_ref[0]] * _NSAMPLE, axis=1)
    y = jnp.dot(w_ref[...], x_ref[0] - sam,
                preferred_element_type=jnp.float32)
    y = y + b_ref[...]
    y_ref[0] = y

    @pl.when(b == 0)
    def _():
        s_ref[...] = jnp.zeros_like(s_ref)
        ss_ref[...] = jnp.zeros_like(ss_ref)

    mc = mc_ref[...]
    s_ref[...] += jnp.dot(y, mc, preferred_element_type=jnp.float32)
    ss_ref[...] += jnp.dot(y * y, mc, preferred_element_type=jnp.float32)


def _l2_kernel(y1_ref, s_ref, ss_ref, mc_ref, g_ref, be_ref, w_ref, b_ref,
               y2_ref, s2_ref, ss2_ref, *, m):
    b = pl.program_id(0)
    mean = s_ref[...] / m
    var = ss_ref[...] / m - mean * mean
    scale = g_ref[...] / jnp.sqrt(var + _EPS)
    shift = be_ref[...] - mean * scale
    h = jnp.maximum(y1_ref[0] * scale + shift, 0.0)
    y = jnp.dot(w_ref[...], h, preferred_element_type=jnp.float32)
    y = y + b_ref[...]
    y2_ref[0] = y

    @pl.when(b == 0)
    def _():
        s2_ref[...] = jnp.zeros_like(s2_ref)
        ss2_ref[...] = jnp.zeros_like(ss2_ref)

    mc = mc_ref[...]
    s2_ref[...] += jnp.dot(y, mc, preferred_element_type=jnp.float32)
    ss2_ref[...] += jnp.dot(y * y, mc, preferred_element_type=jnp.float32)


def _l3_kernel(y2_ref, s_ref, ss_ref, mr_ref, g_ref, be_ref, out_ref, *, m, nw):
    mean = s_ref[...] / m
    var = ss_ref[...] / m - mean * mean
    scale = g_ref[...] / jnp.sqrt(var + _EPS)
    shift = be_ref[...] - mean * scale
    h = jnp.maximum(y2_ref[0] * scale + shift, 0.0)
    h = jnp.where(mr_ref[...] > 0.0, h, -jnp.inf)
    for s in range(_NSAMPLE):
        out_ref[0, :, pl.ds(s, 1)] = jnp.max(
            h[:, s * nw:(s + 1) * nw], axis=1, keepdims=True)


def _mlp(x, sam_pad, maskw, W1, b1, g1, be1, W2, b2, g2, be2):
    B, IC, HW = x.shape
    O1 = W1.shape[0]
    O2 = W2.shape[0]
    m = float(B * _NSAMPLE * 67)

    col = lambda v: v[:, None]
    mask_c = maskw[:, None]
    mask_r = maskw[None, :]

    y1, s1, ss1 = pl.pallas_call(
        _l1_kernel,
        grid=(B,),
        in_specs=[
            pl.BlockSpec((1, IC, HW), lambda b: (b, 0, 0)),
            pl.BlockSpec((1, IC, _FPAD), lambda b: (b, 0, 0)),
            pl.BlockSpec((HW, 1), lambda b: (0, 0)),
            pl.BlockSpec((O1, IC), lambda b: (0, 0)),
            pl.BlockSpec((O1, 1), lambda b: (0, 0)),
        ],
        out_specs=[
            pl.BlockSpec((1, O1, HW), lambda b: (b, 0, 0)),
            pl.BlockSpec((O1, 1), lambda b: (0, 0)),
            pl.BlockSpec((O1, 1), lambda b: (0, 0)),
        ],
        out_shape=[
            jax.ShapeDtypeStruct((B, O1, HW), jnp.float32),
            jax.ShapeDtypeStruct((O1, 1), jnp.float32),
            jax.ShapeDtypeStruct((O1, 1), jnp.float32),
        ],
    )(x, sam_pad, mask_c, W1, col(b1))

    y2, s2, ss2 = pl.pallas_call(
        functools.partial(_l2_kernel, m=m),
        grid=(B,),
        in_specs=[
            pl.BlockSpec((1, O1, HW), lambda b: (b, 0, 0)),
            pl.BlockSpec((O1, 1), lambda b: (0, 0)),
            pl.BlockSpec((O1, 1), lambda b: (0, 0)),
            pl.BlockSpec((HW, 1), lambda b: (0, 0)),
            pl.BlockSpec((O1, 1), lambda b: (0, 0)),
            pl.BlockSpec((O1, 1), lambda b: (0, 0)),
            pl.BlockSpec((O2, O1), lambda b: (0, 0)),
            pl.BlockSpec((O2, 1), lambda b: (0, 0)),
        ],
        out_specs=[
            pl.BlockSpec((1, O2, HW), lambda b: (b, 0, 0)),
            pl.BlockSpec((O2, 1), lambda b: (0, 0)),
            pl.BlockSpec((O2, 1), lambda b: (0, 0)),
        ],
        out_shape=[
            jax.ShapeDtypeStruct((B, O2, HW), jnp.float32),
            jax.ShapeDtypeStruct((O2, 1), jnp.float32),
            jax.ShapeDtypeStruct((O2, 1), jnp.float32),
        ],
    )(y1, s1, ss1, mask_c, col(g1), col(be1), W2, col(b2))

    out = pl.pallas_call(
        functools.partial(_l3_kernel, m=m, nw=_FPAD),
        grid=(B,),
        in_specs=[
            pl.BlockSpec((1, O2, HW), lambda b: (b, 0, 0)),
            pl.BlockSpec((O2, 1), lambda b: (0, 0)),
            pl.BlockSpec((O2, 1), lambda b: (0, 0)),
            pl.BlockSpec((1, HW), lambda b: (0, 0)),
            pl.BlockSpec((O2, 1), lambda b: (0, 0)),
            pl.BlockSpec((O2, 1), lambda b: (0, 0)),
        ],
        out_specs=pl.BlockSpec((1, O2, _NSAMPLE), lambda b: (b, 0, 0)),
        out_shape=jax.ShapeDtypeStruct((B, O2, _NSAMPLE), jnp.float32),
    )(y2, s2, ss2, mask_r, col(g2), col(be2))
    return out


def kernel(xyz, points, W1, b1, g1, be1, W2, b2, g2, be2):
    B, N, C = points.shape
    fps_idx = _fps(xyz)
    bidx = jnp.arange(B)[:, None]
    sampled = xyz[bidx, fps_idx]

    d2 = (jnp.sum(sampled ** 2, -1)[:, :, None] + jnp.sum(xyz ** 2, -1)[:, None, :]
          - 2.0 * jnp.einsum('bpd,bnd->bpn', sampled, xyz))
    fidx = _sc_topk(d2.reshape(B * _NPOINT, N))

    xyz_pad = jnp.pad(xyz, ((0, 0), (0, 0), (0, 13)))
    pts_pad = jnp.pad(points, ((0, 0), (0, 0), (0, _FPAD - 16 - C)))
    table = jnp.concatenate([xyz_pad, pts_pad], axis=2).reshape(B * N, _FPAD)
    rows = _sc_gather(table, fidx)
    x = rows.reshape(B, _NPOINT, _NSAMPLE * _FPAD)

    sam_pad = jnp.pad(sampled, ((0, 0), (0, 0), (0, _FPAD - 3)))
    w = jnp.arange(_FPAD)
    maskw = jnp.tile(((w < 3) | ((w >= 16) & (w < 16 + C))).astype(jnp.float32),
                     _NSAMPLE)

    new_points = _mlp(x, sam_pad, maskw, W1, b1, g1, be1, W2, b2, g2, be2)
    return sampled, new_points
